# Initial kernel scaffold; baseline (speedup 1.0000x reference)
import jax
import jax.numpy as jnp
from jax.experimental import pallas as pl


def kernel(x, embedding):
    raise NotImplementedError("write your pallas kernel here")


# speedup vs baseline: 3.6472x; 3.6472x over previous
import jax
import jax.numpy as jnp
from jax import lax
from jax.experimental import pallas as pl
from jax.experimental.pallas import tpu as pltpu
from jax.experimental.pallas import tpu_sc as plsc

_NC = 2
_NS = 16
_NW = _NC * _NS
_CH = 128


def kernel(x, embedding):
    batch, hist = x.shape
    dim = embedding.shape[1]
    num_indices = batch * hist
    assert num_indices % (_NW * _CH) == 0
    span = num_indices // _NW
    steps = span // _CH

    indices = x.reshape((num_indices,)).astype(jnp.int32)

    mesh = plsc.VectorSubcoreMesh(core_axis_name="c", subcore_axis_name="s")

    @pl.kernel(
        out_type=jax.ShapeDtypeStruct((num_indices, dim), embedding.dtype),
        mesh=mesh,
        compiler_params=pltpu.CompilerParams(use_tc_tiling_on_sc=False),
        scratch_types=[
            pltpu.VMEM((_CH,), jnp.int32),
            pltpu.VMEM((_CH, dim), embedding.dtype),
            pltpu.SemaphoreType.DMA,
        ],
    )
    def gather_kernel(idx_hbm, table_hbm, out_hbm, idx_v, rows_v, sem):
        wid = lax.axis_index("s") * _NC + lax.axis_index("c")
        base = wid * span

        @pl.loop(0, steps)
        def _(i):
            off = base + i * _CH
            pltpu.sync_copy(idx_hbm.at[pl.ds(off, _CH)], idx_v)
            pltpu.async_copy(table_hbm.at[idx_v], rows_v, sem).wait()
            pltpu.sync_copy(rows_v, out_hbm.at[pl.ds(off, _CH)])

    out = gather_kernel(indices, embedding)
    return out.reshape(batch, hist, dim)


# speedup vs baseline: 5.0270x; 1.3783x over previous
import jax
import jax.numpy as jnp
from jax import lax
from jax.experimental import pallas as pl
from jax.experimental.pallas import tpu as pltpu
from jax.experimental.pallas import tpu_sc as plsc

_NC = 2
_NS = 16
_NW = _NC * _NS
_G = 128
_BI = 1024
_NG = _BI // _G


def kernel(x, embedding):
    batch, hist = x.shape
    dim = embedding.shape[1]
    num_indices = batch * hist
    assert num_indices % (_NW * 2 * _BI) == 0
    span = num_indices // _NW
    nblocks = span // _BI
    npairs = nblocks // 2

    indices = x.reshape((num_indices,)).astype(jnp.int32)

    mesh = plsc.VectorSubcoreMesh(core_axis_name="c", subcore_axis_name="s")

    @pl.kernel(
        out_type=jax.ShapeDtypeStruct((num_indices, dim), embedding.dtype),
        mesh=mesh,
        compiler_params=pltpu.CompilerParams(use_tc_tiling_on_sc=False),
        scratch_types=[
            pltpu.VMEM((_BI,), jnp.int32),
            pltpu.VMEM((_BI,), jnp.int32),
            pltpu.VMEM((_BI, dim), embedding.dtype),
            pltpu.VMEM((_BI, dim), embedding.dtype),
            pltpu.SemaphoreType.DMA,
            pltpu.SemaphoreType.DMA,
            pltpu.SemaphoreType.DMA,
            pltpu.SemaphoreType.DMA,
            pltpu.SemaphoreType.DMA,
            pltpu.SemaphoreType.DMA,
        ],
    )
    def gather_kernel(idx_hbm, table_hbm, out_hbm,
                      idx_v0, idx_v1, rows_v0, rows_v1,
                      sem_i0, sem_i1, sem_g0, sem_g1, sem_s0, sem_s1):
        wid = lax.axis_index("s") * _NC + lax.axis_index("c")
        base = wid * span
        idx_v = (idx_v0, idx_v1)
        rows_v = (rows_v0, rows_v1)
        sem_i = (sem_i0, sem_i1)
        sem_g = (sem_g0, sem_g1)
        sem_s = (sem_s0, sem_s1)

        def issue_idx(g, b):
            off = base + jnp.minimum(g, nblocks - 1) * _BI
            pltpu.async_copy(idx_hbm.at[pl.ds(off, _BI)], idx_v[b], sem_i[b])

        def wait_idx(b):
            pltpu.make_async_copy(
                idx_hbm.at[pl.ds(base, _BI)], idx_v[b], sem_i[b]
            ).wait()

        def issue_gathers(b):
            for j in range(_NG):
                sl = pl.ds(j * _G, _G)
                pltpu.async_copy(
                    table_hbm.at[idx_v[b].at[sl]], rows_v[b].at[sl], sem_g[b]
                )

        def drain_gathers(b):
            pltpu.make_async_copy(
                table_hbm.at[pl.ds(0, _BI)], rows_v[b], sem_g[b]
            ).wait()

        def issue_store(g, b):
            off = base + g * _BI
            pltpu.async_copy(rows_v[b], out_hbm.at[pl.ds(off, _BI)], sem_s[b])

        def wait_store(b):
            pltpu.make_async_copy(
                rows_v[b], out_hbm.at[pl.ds(base, _BI)], sem_s[b]
            ).wait()

        issue_idx(0, 0)
        issue_idx(1, 1)
        wait_idx(0)
        issue_gathers(0)
        wait_idx(1)
        issue_gathers(1)
        drain_gathers(0)
        issue_store(0, 0)
        issue_idx(2, 0)
        drain_gathers(1)
        issue_store(1, 1)
        issue_idx(3, 1)

        @pl.loop(1, npairs)
        def _(i):
            g0 = 2 * i
            wait_idx(0)
            wait_store(0)
            issue_gathers(0)
            wait_idx(1)
            wait_store(1)
            issue_gathers(1)
            drain_gathers(0)
            issue_store(g0, 0)
            issue_idx(g0 + 2, 0)
            drain_gathers(1)
            issue_store(g0 + 1, 1)
            issue_idx(g0 + 3, 1)

        wait_idx(0)
        wait_idx(1)
        wait_store(0)
        wait_store(1)

    out = gather_kernel(indices, embedding)
    return out.reshape(batch, hist, dim)


# speedup vs baseline: 5.0286x; 1.0003x over previous
import jax
import jax.numpy as jnp
from jax import lax
from jax.experimental import pallas as pl
from jax.experimental.pallas import tpu as pltpu
from jax.experimental.pallas import tpu_sc as plsc

_NC = 2
_NS = 16
_NW = _NC * _NS
_G = 128
_BI = 1024
_NG = _BI // _G


def kernel(x, embedding):
    batch, hist = x.shape
    dim = embedding.shape[1]
    num_indices = batch * hist
    assert num_indices % (_NW * 2 * _BI) == 0
    span = num_indices // _NW
    nblocks = span // _BI
    npairs = nblocks // 2

    indices = x.reshape((num_indices,)).astype(jnp.int32)

    mesh = plsc.VectorSubcoreMesh(core_axis_name="c", subcore_axis_name="s")

    @pl.kernel(
        out_type=jax.ShapeDtypeStruct((num_indices, dim), embedding.dtype),
        mesh=mesh,
        compiler_params=pltpu.CompilerParams(use_tc_tiling_on_sc=False),
        scratch_types=[
            pltpu.VMEM((_BI,), jnp.int32),
            pltpu.VMEM((_BI,), jnp.int32),
            pltpu.VMEM((_BI, dim), embedding.dtype),
            pltpu.VMEM((_BI, dim), embedding.dtype),
            pltpu.SemaphoreType.DMA,
            pltpu.SemaphoreType.DMA,
            pltpu.SemaphoreType.DMA,
            pltpu.SemaphoreType.DMA,
            pltpu.SemaphoreType.DMA,
            pltpu.SemaphoreType.DMA,
        ],
    )
    def gather_kernel(idx_hbm, table_hbm, out_hbm,
                      idx_v0, idx_v1, rows_v0, rows_v1,
                      sem_i0, sem_i1, sem_g0, sem_g1, sem_s0, sem_s1):
        wid = lax.axis_index("s") * _NC + lax.axis_index("c")
        base = wid * span
        idx_v = (idx_v0, idx_v1)
        rows_v = (rows_v0, rows_v1)
        sem_i = (sem_i0, sem_i1)
        sem_g = (sem_g0, sem_g1)
        sem_s = (sem_s0, sem_s1)

        def issue_idx(g, b):
            off = base + jnp.minimum(g, nblocks - 1) * _BI
            pltpu.async_copy(idx_hbm.at[pl.ds(off, _BI)], idx_v[b], sem_i[b])

        def wait_idx(b):
            pltpu.make_async_copy(
                idx_hbm.at[pl.ds(base, _BI)], idx_v[b], sem_i[b]
            ).wait()

        def issue_gathers(b):
            pltpu.async_copy(table_hbm.at[idx_v[b]], rows_v[b], sem_g[b])

        def drain_gathers(b):
            pltpu.make_async_copy(
                table_hbm.at[pl.ds(0, _BI)], rows_v[b], sem_g[b]
            ).wait()

        def issue_store(g, b):
            off = base + g * _BI
            pltpu.async_copy(rows_v[b], out_hbm.at[pl.ds(off, _BI)], sem_s[b])

        def wait_store(b):
            pltpu.make_async_copy(
                rows_v[b], out_hbm.at[pl.ds(base, _BI)], sem_s[b]
            ).wait()

        issue_idx(0, 0)
        issue_idx(1, 1)
        wait_idx(0)
        issue_gathers(0)
        wait_idx(1)
        issue_gathers(1)
        drain_gathers(0)
        issue_store(0, 0)
        issue_idx(2, 0)
        drain_gathers(1)
        issue_store(1, 1)
        issue_idx(3, 1)

        @pl.loop(1, npairs)
        def _(i):
            g0 = 2 * i
            wait_idx(0)
            wait_store(0)
            issue_gathers(0)
            wait_idx(1)
            wait_store(1)
            issue_gathers(1)
            drain_gathers(0)
            issue_store(g0, 0)
            issue_idx(g0 + 2, 0)
            drain_gathers(1)
            issue_store(g0 + 1, 1)
            issue_idx(g0 + 3, 1)

        wait_idx(0)
        wait_idx(1)
        wait_store(0)
        wait_store(1)

    out = gather_kernel(indices, embedding)
    return out.reshape(batch, hist, dim)
